# Initial kernel scaffold; baseline (speedup 1.0000x reference)
import jax
import jax.numpy as jnp
from jax.experimental import pallas as pl


def kernel(hidden_states, weight):
    raise NotImplementedError("write your pallas kernel here")


# speedup vs baseline: 1.0648x; 1.0648x over previous
import functools

import jax
import jax.numpy as jnp
from jax.experimental import pallas as pl

_N_EXPERTS = 64
_TOP_K = 8
_TILE = 512


def _gate_kernel(x_ref, w_ref, idx_ref, wgt_ref):
    x = x_ref[...]
    w = w_ref[...]
    logits = jax.lax.dot_general(
        x, w, (((1,), (1,)), ((), ())), preferred_element_type=jnp.float32
    )
    t = logits.shape[0]
    lane = jax.lax.broadcasted_iota(jnp.int32, (t, _N_EXPERTS), 1)
    scores = logits
    vals = []
    idxs = []
    for _ in range(_TOP_K):
        m = jnp.max(scores, axis=1, keepdims=True)
        idx = jnp.min(jnp.where(scores == m, lane, _N_EXPERTS), axis=1, keepdims=True)
        vals.append(m)
        idxs.append(idx)
        scores = jnp.where(lane == idx, -jnp.inf, scores)
    top_vals = jnp.concatenate(vals, axis=1)
    top_idx = jnp.concatenate(idxs, axis=1)
    e = jnp.exp(top_vals - top_vals[:, :1])
    wgt_ref[...] = e / jnp.sum(e, axis=1, keepdims=True)
    idx_ref[...] = top_idx


@functools.partial(jax.jit, static_argnums=())
def kernel(hidden_states, weight):
    bsz, seq, h = hidden_states.shape
    tokens = bsz * seq
    x = hidden_states.reshape(tokens, h).astype(jnp.float32)
    w = weight.astype(jnp.float32)
    grid = (tokens // _TILE,)
    idx, wgt = pl.pallas_call(
        _gate_kernel,
        grid=grid,
        in_specs=[
            pl.BlockSpec((_TILE, h), lambda i: (i, 0)),
            pl.BlockSpec((_N_EXPERTS, h), lambda i: (0, 0)),
        ],
        out_specs=[
            pl.BlockSpec((_TILE, _TOP_K), lambda i: (i, 0)),
            pl.BlockSpec((_TILE, _TOP_K), lambda i: (i, 0)),
        ],
        out_shape=[
            jax.ShapeDtypeStruct((tokens, _TOP_K), jnp.int32),
            jax.ShapeDtypeStruct((tokens, _TOP_K), jnp.float32),
        ],
    )(x, w)
    return idx, wgt
